# Initial kernel scaffold; baseline (speedup 1.0000x reference)
import functools

import jax
import jax.numpy as jnp
from jax import lax
from jax.experimental import pallas as pl
from jax.experimental.pallas import tpu as pltpu
from jax.experimental.pallas import tpu_sc as plsc


def _proj_body(w_ref, x_ref, b_ref, o_ref):
    o_ref[...] = lax.dot_general(
        w_ref[...], x_ref[...],
        dimension_numbers=(((1,), (1,)), ((), ())),
        preferred_element_type=jnp.float32,
    ) + b_ref[...]


def kernel(x, edge_index, edge_weight, W, b):
    n, d = x.shape
    e = edge_index.shape[1]

    wt = W.reshape(2, d)
    bias = jnp.concatenate([b.astype(jnp.float32),
                            jnp.zeros((1,), jnp.float32)]).reshape(2, 1)

    proj = pl.pallas_call(
        _proj_body,
        out_shape=jax.ShapeDtypeStruct((2, n), jnp.float32),
    )(wt, x, bias)

    ei = edge_index.astype(jnp.int32)
    ew = edge_weight.astype(jnp.float32)

    info = plsc.get_sparse_core_info()
    nc, ns, lanes = info.num_cores, info.num_subcores, info.num_lanes
    nw = nc * ns
    chunk = e // nw

    @functools.partial(
        pl.kernel,
        mesh=plsc.VectorSubcoreMesh(core_axis_name="c", subcore_axis_name="s"),
        out_type=jax.ShapeDtypeStruct((e,), jnp.float32),
        scratch_types=[
            pltpu.VMEM((n,), jnp.float32),
            pltpu.VMEM((n,), jnp.float32),
            pltpu.VMEM((chunk,), jnp.int32),
            pltpu.VMEM((chunk,), jnp.int32),
            pltpu.VMEM((chunk,), jnp.float32),
            pltpu.VMEM((chunk,), jnp.float32),
        ],
    )
    def edge_scores(p_hbm, ei_hbm, ew_hbm, out_hbm,
                    p1_v, p2_v, src_v, dst_v, ew_v, out_v):
        cid = lax.axis_index("c")
        sid = lax.axis_index("s")
        wid = sid * nc + cid
        base = wid * chunk

        pltpu.sync_copy(p_hbm.at[0], p1_v)
        pltpu.sync_copy(p_hbm.at[1], p2_v)
        pltpu.sync_copy(ei_hbm.at[0, pl.ds(base, chunk)], src_v)
        pltpu.sync_copy(ei_hbm.at[1, pl.ds(base, chunk)], dst_v)
        pltpu.sync_copy(ew_hbm.at[pl.ds(base, chunk)], ew_v)

        def body(i, carry):
            off = pl.multiple_of(i * lanes, lanes)
            s = src_v[pl.ds(off, lanes)]
            t = dst_v[pl.ds(off, lanes)]
            z = plsc.load_gather(p1_v, [s]) + plsc.load_gather(p2_v, [t])
            sig = 1.0 / (1.0 + jnp.exp(-z))
            out_v[pl.ds(off, lanes)] = ew_v[pl.ds(off, lanes)] * sig
            return carry

        lax.fori_loop(0, chunk // lanes, body, 0)
        pltpu.sync_copy(out_v, out_hbm.at[pl.ds(base, chunk)])

    return edge_scores(proj, ei, ew)


# speedup vs baseline: 29.1283x; 29.1283x over previous
import functools

import jax
import jax.numpy as jnp
from jax import lax
from jax.experimental import pallas as pl
from jax.experimental.pallas import tpu as pltpu
from jax.experimental.pallas import tpu_sc as plsc


def _proj_body(w_ref, x_ref, b_ref, o_ref):
    o_ref[...] = lax.dot_general(
        w_ref[...], x_ref[...],
        dimension_numbers=(((1,), (1,)), ((), ())),
        preferred_element_type=jnp.float32,
    ) + b_ref[...]


def kernel(x, edge_index, edge_weight, W, b):
    n, d = x.shape
    e = edge_index.shape[1]

    wt = W.reshape(2, d)
    bias = jnp.concatenate([b.astype(jnp.float32),
                            jnp.zeros((1,), jnp.float32)]).reshape(2, 1)

    proj = pl.pallas_call(
        _proj_body,
        out_shape=jax.ShapeDtypeStruct((2, n), jnp.float32),
    )(wt, x, bias)

    ei = edge_index.astype(jnp.int32)
    src = ei[0]
    dst = ei[1]
    p1 = proj[0]
    p2 = proj[1]
    ew = edge_weight.astype(jnp.float32)

    info = plsc.get_sparse_core_info()
    nc, ns, lanes = info.num_cores, info.num_subcores, info.num_lanes
    nw = nc * ns
    chunk = e // nw

    @functools.partial(
        pl.kernel,
        mesh=plsc.VectorSubcoreMesh(core_axis_name="c", subcore_axis_name="s"),
        out_type=jax.ShapeDtypeStruct((e,), jnp.float32),
        compiler_params=pltpu.CompilerParams(needs_layout_passes=False),
        scratch_types=[
            pltpu.VMEM((n,), jnp.float32),
            pltpu.VMEM((n,), jnp.float32),
            pltpu.VMEM((chunk,), jnp.int32),
            pltpu.VMEM((chunk,), jnp.int32),
            pltpu.VMEM((chunk,), jnp.float32),
            pltpu.VMEM((chunk,), jnp.float32),
        ],
    )
    def edge_scores(p1_hbm, p2_hbm, src_hbm, dst_hbm, ew_hbm, out_hbm,
                    p1_v, p2_v, src_v, dst_v, ew_v, out_v):
        cid = lax.axis_index("c")
        sid = lax.axis_index("s")
        wid = sid * nc + cid
        base = wid * chunk

        pltpu.sync_copy(p1_hbm, p1_v)
        pltpu.sync_copy(p2_hbm, p2_v)
        pltpu.sync_copy(src_hbm.at[pl.ds(base, chunk)], src_v)
        pltpu.sync_copy(dst_hbm.at[pl.ds(base, chunk)], dst_v)
        pltpu.sync_copy(ew_hbm.at[pl.ds(base, chunk)], ew_v)

        def body(i, carry):
            off = pl.multiple_of(i * lanes, lanes)
            s = src_v[pl.ds(off, lanes)]
            t = dst_v[pl.ds(off, lanes)]
            z = plsc.load_gather(p1_v, [s]) + plsc.load_gather(p2_v, [t])
            sig = 1.0 / (1.0 + jnp.exp(-z))
            out_v[pl.ds(off, lanes)] = ew_v[pl.ds(off, lanes)] * sig
            return carry

        lax.fori_loop(0, chunk // lanes, body, 0)
        pltpu.sync_copy(out_v, out_hbm.at[pl.ds(base, chunk)])

    return edge_scores(p1, p2, src, dst, ew)


# speedup vs baseline: 50.2982x; 1.7268x over previous
import functools

import jax
import jax.numpy as jnp
from jax import lax
from jax.experimental import pallas as pl
from jax.experimental.pallas import tpu as pltpu
from jax.experimental.pallas import tpu_sc as plsc


def _proj_body(w_ref, x_ref, b_ref, p1_ref, p2_ref):
    r = lax.dot_general(
        w_ref[...], x_ref[...],
        dimension_numbers=(((1,), (1,)), ((), ())),
        preferred_element_type=jnp.float32,
    )
    p1_ref[...] = r[0] + b_ref[0, 0]
    p2_ref[...] = r[1]


def kernel(x, edge_index, edge_weight, W, b):
    n, d = x.shape
    e = edge_index.shape[1]

    wt = W.reshape(2, d)
    bias = b.astype(jnp.float32).reshape(1, 1)

    p1, p2 = pl.pallas_call(
        _proj_body,
        out_shape=[jax.ShapeDtypeStruct((n,), jnp.float32),
                   jax.ShapeDtypeStruct((n,), jnp.float32)],
    )(wt, x, bias)

    eif = edge_index.astype(jnp.int32).reshape(2 * e)
    ew = edge_weight.astype(jnp.float32)

    info = plsc.get_sparse_core_info()
    nc, ns, lanes = info.num_cores, info.num_subcores, info.num_lanes
    nw = nc * ns
    chunk = e // nw

    @functools.partial(
        pl.kernel,
        mesh=plsc.VectorSubcoreMesh(core_axis_name="c", subcore_axis_name="s"),
        out_type=jax.ShapeDtypeStruct((e,), jnp.float32),
        compiler_params=pltpu.CompilerParams(needs_layout_passes=False),
        scratch_types=[
            pltpu.VMEM((n,), jnp.float32),
            pltpu.VMEM((n,), jnp.float32),
            pltpu.VMEM((chunk,), jnp.int32),
            pltpu.VMEM((chunk,), jnp.int32),
            pltpu.VMEM((chunk,), jnp.float32),
            pltpu.VMEM((chunk,), jnp.float32),
            pltpu.SemaphoreType.DMA,
        ],
    )
    def edge_scores(p1_hbm, p2_hbm, ei_hbm, ew_hbm, out_hbm,
                    p1_v, p2_v, src_v, dst_v, ew_v, out_v, sem):
        cid = lax.axis_index("c")
        sid = lax.axis_index("s")
        wid = sid * nc + cid
        base = wid * chunk

        copies = [
            pltpu.async_copy(p1_hbm, p1_v, sem),
            pltpu.async_copy(p2_hbm, p2_v, sem),
            pltpu.async_copy(ei_hbm.at[pl.ds(base, chunk)], src_v, sem),
            pltpu.async_copy(ei_hbm.at[pl.ds(e + base, chunk)], dst_v, sem),
            pltpu.async_copy(ew_hbm.at[pl.ds(base, chunk)], ew_v, sem),
        ]
        for c in copies:
            c.wait()

        @plsc.parallel_loop(0, chunk, step=lanes, unroll=8)
        def body(off):
            s = src_v[pl.ds(off, lanes)]
            t = dst_v[pl.ds(off, lanes)]
            z = plsc.load_gather(p1_v, [s]) + plsc.load_gather(p2_v, [t])
            sig = 1.0 / (1.0 + jnp.exp(-z))
            out_v[pl.ds(off, lanes)] = ew_v[pl.ds(off, lanes)] * sig

        pltpu.sync_copy(out_v, out_hbm.at[pl.ds(base, chunk)])

    return edge_scores(p1, p2, eif, ew)


# speedup vs baseline: 50.4340x; 1.0027x over previous
import functools

import jax
import jax.numpy as jnp
from jax import lax
from jax.experimental import pallas as pl
from jax.experimental.pallas import tpu as pltpu
from jax.experimental.pallas import tpu_sc as plsc


def _proj_body(w_ref, x_ref, b_ref, p1_ref, p2_ref):
    r = lax.dot_general(
        w_ref[...], x_ref[...],
        dimension_numbers=(((1,), (1,)), ((), ())),
        preferred_element_type=jnp.float32,
    )
    p1_ref[...] = r[0] + b_ref[0, 0]
    p2_ref[...] = r[1]


def kernel(x, edge_index, edge_weight, W, b):
    n, d = x.shape
    e = edge_index.shape[1]

    wt = W.reshape(2, d)
    bias = b.astype(jnp.float32).reshape(1, 1)

    p1, p2 = pl.pallas_call(
        _proj_body,
        out_shape=[jax.ShapeDtypeStruct((n,), jnp.float32),
                   jax.ShapeDtypeStruct((n,), jnp.float32)],
    )(wt, x, bias)

    ei = edge_index.astype(jnp.int32)
    ew = edge_weight.astype(jnp.float32)

    info = plsc.get_sparse_core_info()
    nc, ns, lanes = info.num_cores, info.num_subcores, info.num_lanes
    nw = nc * ns
    chunk = e // nw

    @functools.partial(
        pl.kernel,
        mesh=plsc.VectorSubcoreMesh(core_axis_name="c", subcore_axis_name="s"),
        out_type=jax.ShapeDtypeStruct((e,), jnp.float32),
        compiler_params=pltpu.CompilerParams(needs_layout_passes=False,
                                             use_tc_tiling_on_sc=False),
        scratch_types=[
            pltpu.VMEM((n,), jnp.float32),
            pltpu.VMEM((n,), jnp.float32),
            pltpu.VMEM((chunk,), jnp.int32),
            pltpu.VMEM((chunk,), jnp.int32),
            pltpu.VMEM((chunk,), jnp.float32),
            pltpu.VMEM((chunk,), jnp.float32),
            pltpu.SemaphoreType.DMA,
        ],
    )
    def edge_scores(p1_hbm, p2_hbm, ei_hbm, ew_hbm, out_hbm,
                    p1_v, p2_v, src_v, dst_v, ew_v, out_v, sem):
        cid = lax.axis_index("c")
        sid = lax.axis_index("s")
        wid = sid * nc + cid
        base = wid * chunk

        copies = [
            pltpu.async_copy(p1_hbm, p1_v, sem),
            pltpu.async_copy(p2_hbm, p2_v, sem),
            pltpu.async_copy(ei_hbm.at[0, pl.ds(base, chunk)], src_v, sem),
            pltpu.async_copy(ei_hbm.at[1, pl.ds(base, chunk)], dst_v, sem),
            pltpu.async_copy(ew_hbm.at[pl.ds(base, chunk)], ew_v, sem),
        ]
        for c in copies:
            c.wait()

        @plsc.parallel_loop(0, chunk, step=lanes, unroll=8)
        def body(off):
            s = src_v[pl.ds(off, lanes)]
            t = dst_v[pl.ds(off, lanes)]
            z = plsc.load_gather(p1_v, [s]) + plsc.load_gather(p2_v, [t])
            sig = 1.0 / (1.0 + jnp.exp(-z))
            out_v[pl.ds(off, lanes)] = ew_v[pl.ds(off, lanes)] * sig

        pltpu.sync_copy(out_v, out_hbm.at[pl.ds(base, chunk)])

    return edge_scores(p1, p2, ei, ew)


# speedup vs baseline: 52.0626x; 1.0323x over previous
import functools

import jax
import jax.numpy as jnp
from jax import lax
from jax.experimental import pallas as pl
from jax.experimental.pallas import tpu as pltpu
from jax.experimental.pallas import tpu_sc as plsc


def _prep_body(w_ref, x_ref, b_ref, ei_ref, pk_ref, src_ref, dst_ref):
    @pl.when(pl.program_id(0) == 0)
    def _():
        r = lax.dot_general(
            w_ref[...], x_ref[...],
            dimension_numbers=(((1,), (1,)), ((), ())),
            preferred_element_type=jnp.float32,
        )
        u0 = lax.bitcast_convert_type(r[0] + b_ref[0, 0], jnp.uint32)
        u1 = lax.bitcast_convert_type(r[1], jnp.uint32)
        half = jnp.uint32(0x8000)
        pk = ((u0 + half) & jnp.uint32(0xFFFF0000)) | ((u1 + half) >> 16)
        pk_ref[...] = lax.bitcast_convert_type(pk, jnp.int32)

    src_ref[...] = ei_ref[0]
    dst_ref[...] = ei_ref[1]


def kernel(x, edge_index, edge_weight, W, b):
    n, d = x.shape
    e = edge_index.shape[1]

    wt = W.reshape(2, d)
    bias = b.astype(jnp.float32).reshape(1, 1)
    ei = edge_index.astype(jnp.int32)
    ew = edge_weight.astype(jnp.float32)

    eb = 32768
    steps = -(-e // eb)
    ep = steps * eb

    pk, src, dst = pl.pallas_call(
        _prep_body,
        grid=(steps,),
        in_specs=[
            pl.BlockSpec((2, d), lambda i: (0, 0)),
            pl.BlockSpec((n, d), lambda i: (0, 0)),
            pl.BlockSpec((1, 1), lambda i: (0, 0)),
            pl.BlockSpec((2, eb), lambda i: (0, i)),
        ],
        out_specs=[
            pl.BlockSpec((n,), lambda i: (0,)),
            pl.BlockSpec((eb,), lambda i: (i,)),
            pl.BlockSpec((eb,), lambda i: (i,)),
        ],
        out_shape=[jax.ShapeDtypeStruct((n,), jnp.int32),
                   jax.ShapeDtypeStruct((ep,), jnp.int32),
                   jax.ShapeDtypeStruct((ep,), jnp.int32)],
    )(wt, x, bias, ei)

    info = plsc.get_sparse_core_info()
    nc, ns, lanes = info.num_cores, info.num_subcores, info.num_lanes
    nw = nc * ns
    chunk = e // nw
    ha = ((chunk // 2 + lanes - 1) // lanes) * lanes
    hb = chunk - ha

    @functools.partial(
        pl.kernel,
        mesh=plsc.VectorSubcoreMesh(core_axis_name="c", subcore_axis_name="s"),
        out_type=jax.ShapeDtypeStruct((e,), jnp.float32),
        compiler_params=pltpu.CompilerParams(needs_layout_passes=False,
                                             use_tc_tiling_on_sc=False),
        scratch_types=[
            pltpu.VMEM((n,), jnp.int32),
            pltpu.VMEM((chunk,), jnp.int32),
            pltpu.VMEM((chunk,), jnp.int32),
            pltpu.VMEM((chunk,), jnp.float32),
            pltpu.VMEM((chunk,), jnp.float32),
            pltpu.SemaphoreType.DMA,
            pltpu.SemaphoreType.DMA,
        ],
    )
    def edge_scores(pk_hbm, src_hbm, dst_hbm, ew_hbm, out_hbm,
                    pk_v, src_v, dst_v, ew_v, out_v, sem, sem_out):
        cid = lax.axis_index("c")
        sid = lax.axis_index("s")
        wid = sid * nc + cid
        base = wid * chunk

        hi_mask = jnp.int32(-65536)

        copies_a = [
            pltpu.async_copy(pk_hbm, pk_v, sem),
            pltpu.async_copy(src_hbm.at[pl.ds(base, ha)],
                             src_v.at[pl.ds(0, ha)], sem),
            pltpu.async_copy(dst_hbm.at[pl.ds(base, ha)],
                             dst_v.at[pl.ds(0, ha)], sem),
            pltpu.async_copy(ew_hbm.at[pl.ds(base, ha)],
                             ew_v.at[pl.ds(0, ha)], sem),
        ]
        copies_b = [
            pltpu.async_copy(src_hbm.at[pl.ds(base + ha, hb)],
                             src_v.at[pl.ds(ha, hb)], sem),
            pltpu.async_copy(dst_hbm.at[pl.ds(base + ha, hb)],
                             dst_v.at[pl.ds(ha, hb)], sem),
            pltpu.async_copy(ew_hbm.at[pl.ds(base + ha, hb)],
                             ew_v.at[pl.ds(ha, hb)], sem),
        ]

        def compute(lo, hi):
            @plsc.parallel_loop(lo, hi, step=lanes, unroll=8)
            def body(off):
                s = src_v[pl.ds(off, lanes)]
                t = dst_v[pl.ds(off, lanes)]
                g1 = plsc.load_gather(pk_v, [s])
                g2 = plsc.load_gather(pk_v, [t])
                p1 = plsc.bitcast(g1 & hi_mask, jnp.float32)
                p2 = plsc.bitcast(g2 << 16, jnp.float32)
                z = p1 + p2
                sig = 1.0 / (1.0 + jnp.exp(-z))
                out_v[pl.ds(off, lanes)] = ew_v[pl.ds(off, lanes)] * sig

        for c in copies_a:
            c.wait()
        compute(0, ha)
        out_a = pltpu.async_copy(out_v.at[pl.ds(0, ha)],
                                 out_hbm.at[pl.ds(base, ha)], sem_out)
        for c in copies_b:
            c.wait()
        compute(ha, chunk)
        out_b = pltpu.async_copy(out_v.at[pl.ds(ha, hb)],
                                 out_hbm.at[pl.ds(base + ha, hb)], sem_out)
        out_a.wait()
        out_b.wait()

    return edge_scores(pk, src, dst, ew)


# speedup vs baseline: 61.2537x; 1.1765x over previous
import functools

import jax
import jax.numpy as jnp
from jax import lax
from jax.experimental import pallas as pl
from jax.experimental.pallas import tpu as pltpu
from jax.experimental.pallas import tpu_sc as plsc


def _pack_body(w_ref, x_ref, b_ref, pk_ref):
    r = lax.dot_general(
        w_ref[...], x_ref[...],
        dimension_numbers=(((1,), (1,)), ((), ())),
        preferred_element_type=jnp.float32,
    )
    u0 = lax.bitcast_convert_type(r[0] + b_ref[0, 0], jnp.uint32)
    u1 = lax.bitcast_convert_type(r[1], jnp.uint32)
    half = jnp.uint32(0x8000)
    pk = ((u0 + half) & jnp.uint32(0xFFFF0000)) | ((u1 + half) >> 16)
    pk_ref[...] = lax.bitcast_convert_type(pk, jnp.int32)


def kernel(x, edge_index, edge_weight, W, b):
    n, d = x.shape
    e = edge_index.shape[1]

    wt = W.reshape(2, d)
    bias = b.astype(jnp.float32).reshape(1, 1)
    ei = edge_index.astype(jnp.int32)
    ew = edge_weight.astype(jnp.float32)

    pk = pl.pallas_call(
        _pack_body,
        out_shape=jax.ShapeDtypeStruct((n,), jnp.int32),
    )(wt, x, bias)

    info = plsc.get_sparse_core_info()
    nc, ns, lanes = info.num_cores, info.num_subcores, info.num_lanes
    nw = nc * ns
    blk = 128
    nblk = e // blk
    nb_lo = nblk // nw
    n_hi = nblk - nb_lo * nw
    l_hi = (nb_lo + 1) * blk
    l_lo = nb_lo * blk

    @functools.partial(
        pl.kernel,
        mesh=plsc.VectorSubcoreMesh(core_axis_name="c", subcore_axis_name="s"),
        out_type=jax.ShapeDtypeStruct((e,), jnp.float32),
        compiler_params=pltpu.CompilerParams(needs_layout_passes=False,
                                             use_tc_tiling_on_sc=True),
        scratch_types=[
            pltpu.VMEM((n,), jnp.int32),
            pltpu.VMEM((2, l_hi), jnp.int32),
            pltpu.VMEM((l_hi,), jnp.float32),
            pltpu.VMEM((l_hi,), jnp.float32),
            pltpu.SemaphoreType.DMA,
            pltpu.SemaphoreType.DMA,
        ],
    )
    def edge_scores(pk_hbm, ei_hbm, ew_hbm, out_hbm,
                    pk_v, ei_v, ew_v, out_v, sem, sem_out):
        cid = lax.axis_index("c")
        sid = lax.axis_index("s")
        wid = sid * nc + cid

        hi_mask = jnp.int32(-65536)

        def run(nb):
            ln = nb * blk
            c0 = pl.multiple_of(
                (nb_lo * wid + jnp.minimum(wid, n_hi)) * blk, blk)
            copies = [
                pltpu.async_copy(pk_hbm, pk_v, sem),
                pltpu.async_copy(ei_hbm.at[:, pl.ds(c0, ln)],
                                 ei_v.at[:, pl.ds(0, ln)], sem),
                pltpu.async_copy(ew_hbm.at[pl.ds(c0, ln)],
                                 ew_v.at[pl.ds(0, ln)], sem),
            ]
            for c in copies:
                c.wait()

            @plsc.parallel_loop(0, ln, step=lanes, unroll=8)
            def body(off):
                s = ei_v[0, pl.ds(off, lanes)]
                t = ei_v[1, pl.ds(off, lanes)]
                g1 = plsc.load_gather(pk_v, [s])
                g2 = plsc.load_gather(pk_v, [t])
                p1 = plsc.bitcast(g1 & hi_mask, jnp.float32)
                p2 = plsc.bitcast(g2 << 16, jnp.float32)
                z = p1 + p2
                sig = 1.0 / (1.0 + jnp.exp(-z))
                out_v[pl.ds(off, lanes)] = ew_v[pl.ds(off, lanes)] * sig

            pltpu.async_copy(out_v.at[pl.ds(0, ln)],
                             out_hbm.at[pl.ds(c0, ln)], sem_out).wait()

        @pl.when(wid < n_hi)
        def _():
            run(nb_lo + 1)

        @pl.when(wid >= n_hi)
        def _():
            run(nb_lo)

    return edge_scores(pk, ei, ew)
